# baseline (speedup 1.0000x reference)
import jax
import jax.numpy as jnp
from jax import lax
from jax.experimental import pallas as pl
from jax.experimental.pallas import tpu as pltpu
from jax.experimental.pallas import tpu_sc as plsc

_VOCAB = 100000
_EMB = 1024
_SCALE = 32.0
_NC = 2
_NS = 16
_NW = _NC * _NS
_B_TOT = 4 * 4096
_B_PER_W = _B_TOT // _NW
_CHUNK = 40
_SIZES = [_CHUNK] * (_B_PER_W // _CHUNK) + (
    [_B_PER_W % _CHUNK] if _B_PER_W % _CHUNK else [])
_OFFS = [i * _CHUNK for i in range(len(_SIZES))]
_NCHUNK = len(_SIZES)
_NBUF = 3
_LANES = 16
_VPR = _EMB // _LANES


def _emb_body(tokens_hbm, table_hbm, out_hbm, idx_v, rows_v, gsem, wsem):
    wid = lax.axis_index("s") * _NC + lax.axis_index("c")
    base = wid * _B_PER_W
    pltpu.sync_copy(tokens_hbm.at[pl.ds(base, _CHUNK)], idx_v.at[pl.ds(0, _CHUNK)])

    def start_gather(g):
        b = g % _NBUF
        return pltpu.async_copy(
            table_hbm.at[idx_v.at[pl.ds(_OFFS[g], _SIZES[g])]],
            rows_v.at[b, pl.ds(0, _SIZES[g])],
            gsem,
        )

    gh = {0: start_gather(0)}
    pltpu.sync_copy(
        tokens_hbm.at[pl.ds(base + _CHUNK, _B_PER_W - _CHUNK)],
        idx_v.at[pl.ds(_CHUNK, _B_PER_W - _CHUNK)],
    )
    for g in range(1, _NBUF):
        gh[g] = start_gather(g)
    wh = {}
    for g in range(_NCHUNK):
        b = g % _NBUF
        nxt = g + 1
        if _NBUF <= nxt < _NCHUNK:
            wh[nxt - _NBUF].wait()
            gh[nxt] = start_gather(nxt)
        gh[g].wait()

        def row_body(r, carry, _b=b):
            for c in range(_VPR):
                sl = (_b, r, pl.ds(c * _LANES, _LANES))
                rows_v[sl] = rows_v[sl] * _SCALE
            return carry

        lax.fori_loop(0, _SIZES[g], row_body, 0)
        wh[g] = pltpu.async_copy(
            rows_v.at[b, pl.ds(0, _SIZES[g])],
            out_hbm.at[pl.ds(base + _OFFS[g], _SIZES[g])],
            wsem,
        )
    for g in range(max(0, _NCHUNK - _NBUF), _NCHUNK):
        wh[g].wait()


_emb_kernel = pl.kernel(
    _emb_body,
    out_type=jax.ShapeDtypeStruct((_B_TOT, _EMB), jnp.float32),
    mesh=plsc.VectorSubcoreMesh(
        core_axis_name="c", subcore_axis_name="s",
        num_cores=_NC, num_subcores=_NS,
    ),
    scratch_types=[
        pltpu.VMEM((_B_PER_W,), jnp.int32),
        pltpu.VMEM((_NBUF, _CHUNK, _EMB), jnp.float32),
        pltpu.SemaphoreType.DMA,
        pltpu.SemaphoreType.DMA,
    ],
)


def kernel(tokens, table):
    b, s = tokens.shape
    flat = jnp.reshape(tokens.astype(jnp.int32), (b * s,))
    out = _emb_kernel(flat, table)
    return jnp.reshape(out, (b, s, _EMB))


# speedup vs baseline: 1.0332x; 1.0332x over previous
import jax
import jax.numpy as jnp
from jax import lax
from jax.experimental import pallas as pl
from jax.experimental.pallas import tpu as pltpu
from jax.experimental.pallas import tpu_sc as plsc

_VOCAB = 100000
_EMB = 1024
_SCALE = 32.0
_NC = 2
_NS = 16
_NW = _NC * _NS
_B_TOT = 4 * 4096
_B_PER_W = _B_TOT // _NW
_CHUNK = 32
_SIZES = [_CHUNK] * (_B_PER_W // _CHUNK) + (
    [_B_PER_W % _CHUNK] if _B_PER_W % _CHUNK else [])
_OFFS = [i * _CHUNK for i in range(len(_SIZES))]
_NCHUNK = len(_SIZES)
_NBUF = 3
_LANES = 16
_VPR = _EMB // _LANES


def _emb_body(tokens_hbm, table_hbm, out_hbm, idx_v, rows_v, gsem, wsem):
    wid = lax.axis_index("s") * _NC + lax.axis_index("c")
    base = wid * _B_PER_W
    pltpu.sync_copy(tokens_hbm.at[pl.ds(base, _CHUNK)], idx_v.at[pl.ds(0, _CHUNK)])

    def start_gather(g):
        b = g % _NBUF
        return pltpu.async_copy(
            table_hbm.at[idx_v.at[pl.ds(_OFFS[g], _SIZES[g])]],
            rows_v.at[b, pl.ds(0, _SIZES[g])],
            gsem,
        )

    gh = {0: start_gather(0)}
    pltpu.sync_copy(
        tokens_hbm.at[pl.ds(base + _CHUNK, _B_PER_W - _CHUNK)],
        idx_v.at[pl.ds(_CHUNK, _B_PER_W - _CHUNK)],
    )
    for g in range(1, _NBUF):
        gh[g] = start_gather(g)
    wh = {}
    for g in range(_NCHUNK):
        b = g % _NBUF
        nxt = g + 1
        if _NBUF <= nxt < _NCHUNK:
            wh[nxt - _NBUF].wait()
            gh[nxt] = start_gather(nxt)
        gh[g].wait()

        def row_body(r, carry, _b=b):
            for c in range(_VPR):
                sl = (_b, r, pl.ds(c * _LANES, _LANES))
                rows_v[sl] = rows_v[sl] * _SCALE
            return carry

        lax.fori_loop(0, _SIZES[g], row_body, 0)
        wh[g] = pltpu.async_copy(
            rows_v.at[b, pl.ds(0, _SIZES[g])],
            out_hbm.at[pl.ds(base + _OFFS[g], _SIZES[g])],
            wsem,
        )
    for g in range(max(0, _NCHUNK - _NBUF), _NCHUNK):
        wh[g].wait()


_emb_kernel = pl.kernel(
    _emb_body,
    out_type=jax.ShapeDtypeStruct((_B_TOT, _EMB), jnp.float32),
    mesh=plsc.VectorSubcoreMesh(
        core_axis_name="c", subcore_axis_name="s",
        num_cores=_NC, num_subcores=_NS,
    ),
    scratch_types=[
        pltpu.VMEM((_B_PER_W,), jnp.int32),
        pltpu.VMEM((_NBUF, _CHUNK, _EMB), jnp.float32),
        pltpu.SemaphoreType.DMA,
        pltpu.SemaphoreType.DMA,
    ],
)


def kernel(tokens, table):
    b, s = tokens.shape
    flat = jnp.reshape(tokens.astype(jnp.int32), (b * s,))
    out = _emb_kernel(flat, table)
    return jnp.reshape(out, (b, s, _EMB))
